# baseline (speedup 1.0000x reference)
import functools

import jax
import jax.numpy as jnp
from jax import lax
from jax.experimental import pallas as pl
from jax.experimental.pallas import tpu as pltpu
from jax.experimental.pallas import tpu_sc as plsc

N = 10000
D = 256
DH = 128
E = 160000
P = 65536

NC = 2
NS = 16
L = 16
CH = 128
CH2 = 128

RB = 624
ECH = E // CH
ECH_T = ECH // NS
ECH_R = ECH - ECH_T * NS
ECH_W = ECH // (NC * NS)
ECH_WR = ECH - ECH_W * NC * NS
PCH_T = P // (NC * NS) // CH

_MESH = plsc.VectorSubcoreMesh(
    core_axis_name="c", subcore_axis_name="s", num_cores=NC, num_subcores=NS)


@functools.partial(
    pl.kernel,
    out_type=jax.ShapeDtypeStruct((NC * N,), jnp.float32),
    mesh=_MESH,
    scratch_types=[
        pltpu.VMEM((2, 1, CH), jnp.int32),
        pltpu.VMEM((CH,), jnp.float32),
        pltpu.VMEM((N,), jnp.float32),
        pltpu.VMEM_SHARED((N,), jnp.float32),
        pltpu.SemaphoreType.DMA,
        pltpu.SemaphoreType.DMA,
        pltpu.SemaphoreType.DMA,
        pltpu.SemaphoreType.DMA,
    ],
)
def _sc_deg(dst_hbm, zeros_hbm, deg_out, idx_v, ones_v, deg_v, deg_sh,
            si0, si1, ss0, ss1):
    c = lax.axis_index("c")
    s = lax.axis_index("s")
    wid = c * NS + s
    sem_i = (si0, si1)
    sem_s = (ss0, ss1)

    @pl.when(s == 0)
    def _():
        pltpu.sync_copy(zeros_hbm, deg_sh)

    for j in range(CH // L):
        ones_v[pl.ds(j * L, L)] = jnp.ones((L,), jnp.float32)
    plsc.subcore_barrier()

    n_i = ECH_W + jnp.where(wid < ECH_WR, 1, 0)

    def start_idx(ch, b):
        base = (wid + NC * NS * ch) * CH
        pltpu.async_copy(dst_hbm.at[pl.ds(base, CH)], idx_v.at[b, 0],
                         sem_i[b])

    def start_scatter(b):
        pltpu.make_async_copy(dst_hbm.at[pl.ds(0, CH)], idx_v.at[b, 0],
                              sem_i[b]).wait()
        pltpu.async_copy(ones_v, deg_sh.at[idx_v.at[b, 0]], sem_s[b],
                         add=True)

    def wait_scatter(b):
        pltpu.make_async_copy(ones_v, deg_sh.at[pl.ds(0, CH)],
                              sem_s[b]).wait()

    start_idx(0, 0)
    start_idx(1, 1)

    def body(j, carry):
        c0 = 2 * j
        c1 = 2 * j + 1

        @pl.when((j > 0) & (c0 - 2 < n_i))
        def _():
            wait_scatter(0)

        @pl.when((j > 0) & (c0 < n_i))
        def _():
            start_idx(c0, 0)

        @pl.when(c0 < n_i)
        def _():
            start_scatter(0)

        @pl.when((j > 0) & (c1 - 2 < n_i))
        def _():
            wait_scatter(1)

        @pl.when((j > 0) & (c1 < n_i))
        def _():
            start_idx(c1, 1)

        @pl.when(c1 < n_i)
        def _():
            start_scatter(1)

        return carry

    lax.fori_loop(0, (ECH_W + 1 + 1) // 2, body, 0)
    wait_scatter(0)

    @pl.when(n_i == ECH_W + 1)
    def _():
        wait_scatter(1)

    plsc.subcore_barrier()

    @pl.when(s == 0)
    def _():
        pltpu.sync_copy(deg_sh, deg_v)
        pltpu.sync_copy(deg_v, deg_out.at[pl.ds(c * N, N)])


ECH2 = E // CH2
ECH2_T = ECH2 // NS
ECH2_R = ECH2 - ECH2_T * NS


CHR = CH2 // 128


@functools.partial(
    pl.kernel,
    out_type=(jax.ShapeDtypeStruct((N, DH), jnp.float32),
              jax.ShapeDtypeStruct((N, DH), jnp.float32)),
    mesh=_MESH,
    scratch_types=[
        pltpu.VMEM((2, CHR, 128), jnp.int32),
        pltpu.VMEM((2, CHR, 128), jnp.int32),
        pltpu.VMEM((2, CH2, DH), jnp.float32),
        pltpu.VMEM_SHARED((N, DH), jnp.float32),
        pltpu.SemaphoreType.DMA,
        pltpu.SemaphoreType.DMA,
        pltpu.SemaphoreType.DMA,
        pltpu.SemaphoreType.DMA,
        pltpu.SemaphoreType.DMA,
        pltpu.SemaphoreType.DMA,
        pltpu.SemaphoreType.DMA,
        pltpu.SemaphoreType.DMA,
    ],
)
def _sc_scatter(xa_hbm, xb_hbm, src2_hbm, dst2_hbm, zrow_hbm, out_a, out_b,
                isrc_v, idst_v, rows_v, acc_sh,
                sis0, sis1, sid0, sid1, sg0, sg1, ss0, ss1):
    c = lax.axis_index("c")
    s = lax.axis_index("s")
    base_r = s * RB
    sem_is = (sis0, sis1)
    sem_id = (sid0, sid1)
    sem_g = (sg0, sg1)
    sem_s = (ss0, ss1)

    pltpu.sync_copy(zrow_hbm.at[pl.ds(0, RB)], acc_sh.at[pl.ds(base_r, RB)])

    @pl.when(s == NS - 1)
    def _():
        pltpu.sync_copy(zrow_hbm.at[pl.ds(0, 16)], acc_sh.at[pl.ds(N - 16, 16)])

    plsc.subcore_barrier()

    n_i = ECH2_T + jnp.where(s < ECH2_R, 1, 0)

    def idx_src(ch, b):
        row0 = (s + NS * ch) * CHR
        pltpu.async_copy(src2_hbm.at[pl.ds(row0, CHR)], isrc_v.at[b],
                         sem_is[b])

    def idx_dst(ch, b):
        row0 = (s + NS * ch) * CHR
        pltpu.async_copy(dst2_hbm.at[pl.ds(row0, CHR)], idst_v.at[b],
                         sem_id[b])

    def start_gather(b):
        pltpu.make_async_copy(src2_hbm.at[pl.ds(0, CHR)], isrc_v.at[b],
                              sem_is[b]).wait()

        @pl.when(c == 0)
        def _():
            for r in range(CHR):
                pltpu.async_copy(xa_hbm.at[isrc_v.at[b, r]],
                                 rows_v.at[b, pl.ds(r * CH, CH)], sem_g[b])

        @pl.when(c == 1)
        def _():
            for r in range(CHR):
                pltpu.async_copy(xb_hbm.at[isrc_v.at[b, r]],
                                 rows_v.at[b, pl.ds(r * CH, CH)], sem_g[b])

    def start_scatter(b):
        pltpu.make_async_copy(xa_hbm.at[pl.ds(0, CH2)], rows_v.at[b],
                              sem_g[b]).wait()
        pltpu.make_async_copy(src2_hbm.at[pl.ds(0, CHR)], idst_v.at[b],
                              sem_id[b]).wait()
        for r in range(CHR):
            pltpu.async_copy(rows_v.at[b, pl.ds(r * CH, CH)],
                             acc_sh.at[idst_v.at[b, r]], sem_s[b], add=True)

    def wait_scatter(b):
        pltpu.make_async_copy(rows_v.at[b], acc_sh.at[pl.ds(0, CH2)],
                              sem_s[b]).wait()

    idx_src(0, 0)
    idx_dst(0, 0)
    idx_src(1, 1)
    idx_dst(1, 1)

    def body(j, carry):
        c0 = 2 * j
        c1 = 2 * j + 1

        @pl.when((j > 0) & (c0 - 2 < n_i))
        def _():
            wait_scatter(0)

        @pl.when((j > 0) & (c0 < n_i))
        def _():
            idx_dst(c0, 0)

        @pl.when(c0 < n_i)
        def _():
            start_gather(0)

        @pl.when((j > 0) & (c1 - 2 < n_i))
        def _():
            wait_scatter(1)

        @pl.when((j > 0) & (c1 < n_i))
        def _():
            idx_dst(c1, 1)

        @pl.when(c1 < n_i)
        def _():
            start_gather(1)

        @pl.when(c0 < n_i)
        def _():
            start_scatter(0)

        @pl.when(c0 + 2 < n_i)
        def _():
            idx_src(c0 + 2, 0)

        @pl.when(c1 < n_i)
        def _():
            start_scatter(1)

        @pl.when(c1 + 2 < n_i)
        def _():
            idx_src(c1 + 2, 1)

        return carry

    lax.fori_loop(0, (ECH2_T + ECH2_R + 1) // 2, body, 0)
    @pl.when(n_i == ECH2_T + 1)
    def _():
        wait_scatter(0)

    plsc.subcore_barrier()

    @pl.when(c == 0)
    def _():
        pltpu.sync_copy(acc_sh.at[pl.ds(base_r, RB)],
                        out_a.at[pl.ds(base_r, RB)])

    @pl.when(c == 1)
    def _():
        pltpu.sync_copy(acc_sh.at[pl.ds(base_r, RB)],
                        out_b.at[pl.ds(base_r, RB)])

    @pl.when((c == 0) & (s == NS - 1))
    def _():
        pltpu.sync_copy(acc_sh.at[pl.ds(N - 16, 16)],
                        out_a.at[pl.ds(N - 16, 16)])

    @pl.when((c == 1) & (s == NS - 1))
    def _():
        pltpu.sync_copy(acc_sh.at[pl.ds(N - 16, 16)],
                        out_b.at[pl.ds(N - 16, 16)])


CHP = 64
PCH = P // (NC * NS) // CHP


@functools.partial(
    pl.kernel,
    out_type=jax.ShapeDtypeStruct((P,), jnp.float32),
    mesh=_MESH,
    scratch_types=[
        pltpu.VMEM((2, CHP), jnp.int32),
        pltpu.VMEM((2, CHP), jnp.int32),
        pltpu.VMEM((2, CHP, D), jnp.float32),
        pltpu.VMEM((2, CHP, D), jnp.float32),
        pltpu.VMEM((2, CHP), jnp.float32),
        pltpu.SemaphoreType.DMA,
        pltpu.SemaphoreType.DMA,
        pltpu.SemaphoreType.DMA,
        pltpu.SemaphoreType.DMA,
        pltpu.SemaphoreType.DMA,
        pltpu.SemaphoreType.DMA,
    ],
)
def _sc_pairs(x2_hbm, pa_hbm, pb_hbm, out_hbm, ia_v, ib_v, ra_v, rb_v,
              sc_v, si0, si1, sa0, sa1, sb0, sb1):
    c = lax.axis_index("c")
    s = lax.axis_index("s")
    wid = c * NS + s
    lane = lax.iota(jnp.int32, L)
    sem_i = (si0, si1)
    sem_a = (sa0, sa1)
    sem_b = (sb0, sb1)

    def fetch_gather(ch, b):
        base = wid * (PCH * CHP) + ch * CHP
        pltpu.async_copy(pa_hbm.at[pl.ds(base, CHP)], ia_v.at[b], sem_i[b])
        pltpu.async_copy(pb_hbm.at[pl.ds(base, CHP)], ib_v.at[b], sem_i[b])
        pltpu.make_async_copy(pa_hbm.at[pl.ds(0, CHP)], ia_v.at[b],
                              sem_i[b]).wait()
        pltpu.make_async_copy(pa_hbm.at[pl.ds(0, CHP)], ib_v.at[b],
                              sem_i[b]).wait()
        pltpu.async_copy(x2_hbm.at[ia_v.at[b]], ra_v.at[b], sem_a[b])
        pltpu.async_copy(x2_hbm.at[ib_v.at[b]], rb_v.at[b], sem_b[b])

    perms = [jnp.bitwise_xor(lane, k) for k in (8, 4, 2, 1)]
    masks = [lane == pp for pp in range(L)]

    def compute(ch, b):
        pltpu.make_async_copy(x2_hbm.at[pl.ds(0, CHP)], ra_v.at[b],
                              sem_a[b]).wait()
        pltpu.make_async_copy(x2_hbm.at[pl.ds(0, CHP)], rb_v.at[b],
                              sem_b[b]).wait()

        def butterfly(acc):
            for perm in perms:
                acc = acc + lax.gather(
                    acc, perm[:, None],
                    lax.GatherDimensionNumbers(
                        offset_dims=(), collapsed_slice_dims=(0,),
                        start_index_map=(0,)),
                    slice_sizes=(1,),
                    mode=lax.GatherScatterMode.PROMISE_IN_BOUNDS)
            return acc

        @plsc.parallel_loop(0, CHP // L, 1)
        def qbody(q):
            vec = jnp.zeros((L,), jnp.float32)
            for pp in range(L):
                p = q * L + pp
                z = jnp.zeros((L,), jnp.float32)
                partial = [z, z, z, z]
                for j in range(D // L):
                    partial[j % 4] = partial[j % 4] + (
                        ra_v[b, p, pl.ds(j * L, L)] *
                        rb_v[b, p, pl.ds(j * L, L)])
                acc = butterfly((partial[0] + partial[1]) +
                                (partial[2] + partial[3]))
                vec = jnp.where(masks[pp], acc, vec)
            sc_v[b, pl.ds(q * L, L)] = 1.0 / (1.0 + jnp.exp(-vec))

        base = wid * (PCH * CHP) + ch * CHP
        pltpu.sync_copy(sc_v.at[b], out_hbm.at[pl.ds(base, CHP)])

    fetch_gather(0, 0)
    fetch_gather(1, 1)

    def body(j, carry):
        compute(2 * j, 0)

        @pl.when(j < PCH // 2 - 1)
        def _():
            fetch_gather(2 * j + 2, 0)

        compute(2 * j + 1, 1)

        @pl.when(j < PCH // 2 - 1)
        def _():
            fetch_gather(2 * j + 3, 1)

        return carry

    lax.fori_loop(0, PCH // 2, body, 0)


BN = 2000
GRID = N // BN


def _dinv(degT_ref):
    d = degT_ref[:, 0:1] + degT_ref[:, 1:2] + 1.0
    return lax.rsqrt(d)


def _tc0_body(emb_ref, w1_ref, ha_ref, hb_ref):
    h = jnp.dot(emb_ref[...], w1_ref[...], preferred_element_type=jnp.float32)
    ha_ref[...] = h[:, :DH]
    hb_ref[...] = h[:, DH:]


def _tc1_body(degT_ref, ha_ref, hb_ref, xa_ref, xb_ref):
    dinv = _dinv(degT_ref)
    xa_ref[...] = ha_ref[...] * dinv
    xb_ref[...] = hb_ref[...] * dinv


def _tc2_body(degT_ref, sa_ref, sb_ref, xa_ref, xb_ref, b1_ref, w2_ref,
              ya_ref, yb_ref):
    dinv = _dinv(degT_ref)
    su = jnp.concatenate([sa_ref[...] + xa_ref[...],
                          sb_ref[...] + xb_ref[...]], axis=1)
    x1 = jnp.maximum(dinv * su + b1_ref[...], 0.0)
    h = jnp.dot(x1 * dinv, w2_ref[...], preferred_element_type=jnp.float32)
    ya_ref[...] = h[:, :DH]
    yb_ref[...] = h[:, DH:]


def _tc3_body(degT_ref, sa_ref, sb_ref, xa_ref, xb_ref, b2_ref, x2_ref):
    dinv = _dinv(degT_ref)
    su = jnp.concatenate([sa_ref[...] + xa_ref[...],
                          sb_ref[...] + xb_ref[...]], axis=1)
    x2_ref[...] = dinv * su + b2_ref[...]


def _rows(shape):
    return pl.BlockSpec(shape, lambda i: (i, 0))


def _full(shape):
    return pl.BlockSpec(shape, lambda i: (0, 0))


_tc0 = pl.pallas_call(
    _tc0_body,
    grid=(GRID,),
    in_specs=[_rows((BN, D)), _full((D, D))],
    out_specs=[_rows((BN, DH)), _rows((BN, DH))],
    out_shape=(jax.ShapeDtypeStruct((N, DH), jnp.float32),
               jax.ShapeDtypeStruct((N, DH), jnp.float32)),
)

_tc1 = pl.pallas_call(
    _tc1_body,
    grid=(GRID,),
    in_specs=[_rows((BN, 2)), _rows((BN, DH)), _rows((BN, DH))],
    out_specs=[_rows((BN, DH)), _rows((BN, DH))],
    out_shape=(jax.ShapeDtypeStruct((N, DH), jnp.float32),
               jax.ShapeDtypeStruct((N, DH), jnp.float32)),
)

_tc2 = pl.pallas_call(
    _tc2_body,
    grid=(GRID,),
    in_specs=[_rows((BN, 2)), _rows((BN, DH)), _rows((BN, DH)),
              _rows((BN, DH)), _rows((BN, DH)), _full((1, D)), _full((D, D))],
    out_specs=[_rows((BN, DH)), _rows((BN, DH))],
    out_shape=(jax.ShapeDtypeStruct((N, DH), jnp.float32),
               jax.ShapeDtypeStruct((N, DH), jnp.float32)),
)

_tc3 = pl.pallas_call(
    _tc3_body,
    grid=(GRID,),
    in_specs=[_rows((BN, 2)), _rows((BN, DH)), _rows((BN, DH)),
              _rows((BN, DH)), _rows((BN, DH)), _full((1, D))],
    out_specs=_rows((BN, D)),
    out_shape=jax.ShapeDtypeStruct((N, D), jnp.float32),
)


def kernel(edge_index, edge_pairs, emb, W1, b1, W2, b2):
    src = edge_index[0].astype(jnp.int32)
    dst = edge_index[1].astype(jnp.int32)
    pa = edge_pairs[:, 0].astype(jnp.int32)
    pb = edge_pairs[:, 1].astype(jnp.int32)

    zdeg = jnp.zeros((N,), jnp.float32)
    zrow = jnp.zeros((RB, DH), jnp.float32)

    src2 = src.reshape(E // 128, 128)
    dst2 = dst.reshape(E // 128, 128)

    deg2 = _sc_deg(dst, zdeg)
    degT = deg2.reshape(NC, N).T

    h0a, h0b = _tc0(emb, W1)
    xa0, xb0 = _tc1(degT, h0a, h0b)
    s1a, s1b = _sc_scatter(xa0, xb0, src2, dst2, zrow)
    ya, yb = _tc2(degT, s1a, s1b, xa0, xb0, b1.reshape(1, D), W2)
    s2a, s2b = _sc_scatter(ya, yb, src2, dst2, zrow)
    x2 = _tc3(degT, s2a, s2b, ya, yb, b2.reshape(1, D))

    return _sc_pairs(x2, pa, pb)


# speedup vs baseline: 1.0623x; 1.0623x over previous
import functools

import jax
import jax.numpy as jnp
from jax import lax
from jax.experimental import pallas as pl
from jax.experimental.pallas import tpu as pltpu
from jax.experimental.pallas import tpu_sc as plsc

N = 10000
D = 256
DH = 128
E = 160000
P = 65536

NC = 2
NS = 16
L = 16
CH = 128
CH2 = 128

RB = 624
ECH = E // CH
ECH_T = ECH // NS
ECH_R = ECH - ECH_T * NS
ECH_W = ECH // (NC * NS)
ECH_WR = ECH - ECH_W * NC * NS
PCH_T = P // (NC * NS) // CH

_MESH = plsc.VectorSubcoreMesh(
    core_axis_name="c", subcore_axis_name="s", num_cores=NC, num_subcores=NS)


@functools.partial(
    pl.kernel,
    out_type=jax.ShapeDtypeStruct((NC * N,), jnp.float32),
    mesh=_MESH,
    scratch_types=[
        pltpu.VMEM((2, 1, CH), jnp.int32),
        pltpu.VMEM((CH,), jnp.float32),
        pltpu.VMEM((N,), jnp.float32),
        pltpu.VMEM_SHARED((N,), jnp.float32),
        pltpu.SemaphoreType.DMA,
        pltpu.SemaphoreType.DMA,
        pltpu.SemaphoreType.DMA,
        pltpu.SemaphoreType.DMA,
    ],
)
def _sc_deg(dst_hbm, zeros_hbm, deg_out, idx_v, ones_v, deg_v, deg_sh,
            si0, si1, ss0, ss1):
    c = lax.axis_index("c")
    s = lax.axis_index("s")
    wid = c * NS + s
    sem_i = (si0, si1)
    sem_s = (ss0, ss1)

    @pl.when(s == 0)
    def _():
        pltpu.sync_copy(zeros_hbm, deg_sh)

    for j in range(CH // L):
        ones_v[pl.ds(j * L, L)] = jnp.ones((L,), jnp.float32)
    plsc.subcore_barrier()

    n_i = ECH_W + jnp.where(wid < ECH_WR, 1, 0)

    def start_idx(ch, b):
        base = (wid + NC * NS * ch) * CH
        pltpu.async_copy(dst_hbm.at[pl.ds(base, CH)], idx_v.at[b, 0],
                         sem_i[b])

    def start_scatter(b):
        pltpu.make_async_copy(dst_hbm.at[pl.ds(0, CH)], idx_v.at[b, 0],
                              sem_i[b]).wait()
        pltpu.async_copy(ones_v, deg_sh.at[idx_v.at[b, 0]], sem_s[b],
                         add=True)

    def wait_scatter(b):
        pltpu.make_async_copy(ones_v, deg_sh.at[pl.ds(0, CH)],
                              sem_s[b]).wait()

    start_idx(0, 0)
    start_idx(1, 1)

    def body(j, carry):
        c0 = 2 * j
        c1 = 2 * j + 1

        @pl.when((j > 0) & (c0 - 2 < n_i))
        def _():
            wait_scatter(0)

        @pl.when((j > 0) & (c0 < n_i))
        def _():
            start_idx(c0, 0)

        @pl.when(c0 < n_i)
        def _():
            start_scatter(0)

        @pl.when((j > 0) & (c1 - 2 < n_i))
        def _():
            wait_scatter(1)

        @pl.when((j > 0) & (c1 < n_i))
        def _():
            start_idx(c1, 1)

        @pl.when(c1 < n_i)
        def _():
            start_scatter(1)

        return carry

    lax.fori_loop(0, (ECH_W + 1 + 1) // 2, body, 0)
    wait_scatter(0)

    @pl.when(n_i == ECH_W + 1)
    def _():
        wait_scatter(1)

    plsc.subcore_barrier()

    @pl.when(s == 0)
    def _():
        pltpu.sync_copy(deg_sh, deg_v)
        pltpu.sync_copy(deg_v, deg_out.at[pl.ds(c * N, N)])


ECH2 = E // CH2
ECH2_T = ECH2 // NS
ECH2_R = ECH2 - ECH2_T * NS


CHR = CH2 // 128


@functools.partial(
    pl.kernel,
    out_type=(jax.ShapeDtypeStruct((N, DH), jnp.float32),
              jax.ShapeDtypeStruct((N, DH), jnp.float32)),
    mesh=_MESH,
    scratch_types=[
        pltpu.VMEM((2, CHR, 128), jnp.int32),
        pltpu.VMEM((2, CHR, 128), jnp.int32),
        pltpu.VMEM((2, CH2, DH), jnp.float32),
        pltpu.VMEM_SHARED((N, DH), jnp.float32),
        pltpu.SemaphoreType.DMA,
        pltpu.SemaphoreType.DMA,
        pltpu.SemaphoreType.DMA,
        pltpu.SemaphoreType.DMA,
        pltpu.SemaphoreType.DMA,
        pltpu.SemaphoreType.DMA,
        pltpu.SemaphoreType.DMA,
        pltpu.SemaphoreType.DMA,
    ],
)
def _sc_scatter(xa_hbm, xb_hbm, src2_hbm, dst2_hbm, zrow_hbm, out_a, out_b,
                isrc_v, idst_v, rows_v, acc_sh,
                sis0, sis1, sid0, sid1, sg0, sg1, ss0, ss1):
    c = lax.axis_index("c")
    s = lax.axis_index("s")
    base_r = s * RB
    sem_is = (sis0, sis1)
    sem_id = (sid0, sid1)
    sem_g = (sg0, sg1)
    sem_s = (ss0, ss1)

    pltpu.sync_copy(zrow_hbm.at[pl.ds(0, RB)], acc_sh.at[pl.ds(base_r, RB)])

    @pl.when(s == NS - 1)
    def _():
        pltpu.sync_copy(zrow_hbm.at[pl.ds(0, 16)], acc_sh.at[pl.ds(N - 16, 16)])

    plsc.subcore_barrier()

    n_i = ECH2_T + jnp.where(s < ECH2_R, 1, 0)

    def idx_src(ch, b):
        row0 = (s + NS * ch) * CHR
        pltpu.async_copy(src2_hbm.at[pl.ds(row0, CHR)], isrc_v.at[b],
                         sem_is[b])

    def idx_dst(ch, b):
        row0 = (s + NS * ch) * CHR
        pltpu.async_copy(dst2_hbm.at[pl.ds(row0, CHR)], idst_v.at[b],
                         sem_id[b])

    def start_gather(b):
        pltpu.make_async_copy(src2_hbm.at[pl.ds(0, CHR)], isrc_v.at[b],
                              sem_is[b]).wait()

        @pl.when(c == 0)
        def _():
            for r in range(CHR):
                pltpu.async_copy(xa_hbm.at[isrc_v.at[b, r]],
                                 rows_v.at[b, pl.ds(r * CH, CH)], sem_g[b])

        @pl.when(c == 1)
        def _():
            for r in range(CHR):
                pltpu.async_copy(xb_hbm.at[isrc_v.at[b, r]],
                                 rows_v.at[b, pl.ds(r * CH, CH)], sem_g[b])

    def start_scatter(b):
        pltpu.make_async_copy(xa_hbm.at[pl.ds(0, CH2)], rows_v.at[b],
                              sem_g[b]).wait()
        pltpu.make_async_copy(src2_hbm.at[pl.ds(0, CHR)], idst_v.at[b],
                              sem_id[b]).wait()
        for r in range(CHR):
            pltpu.async_copy(rows_v.at[b, pl.ds(r * CH, CH)],
                             acc_sh.at[idst_v.at[b, r]], sem_s[b], add=True)

    def wait_scatter(b):
        pltpu.make_async_copy(rows_v.at[b], acc_sh.at[pl.ds(0, CH2)],
                              sem_s[b]).wait()

    idx_src(0, 0)
    idx_dst(0, 0)
    idx_src(1, 1)
    idx_dst(1, 1)

    def body(j, carry):
        c0 = 2 * j
        c1 = 2 * j + 1

        @pl.when((j > 0) & (c0 - 2 < n_i))
        def _():
            wait_scatter(0)

        @pl.when((j > 0) & (c0 < n_i))
        def _():
            idx_dst(c0, 0)

        @pl.when(c0 < n_i)
        def _():
            start_gather(0)

        @pl.when((j > 0) & (c1 - 2 < n_i))
        def _():
            wait_scatter(1)

        @pl.when((j > 0) & (c1 < n_i))
        def _():
            idx_dst(c1, 1)

        @pl.when(c1 < n_i)
        def _():
            start_gather(1)

        @pl.when(c0 < n_i)
        def _():
            start_scatter(0)

        @pl.when(c0 + 2 < n_i)
        def _():
            idx_src(c0 + 2, 0)

        @pl.when(c1 < n_i)
        def _():
            start_scatter(1)

        @pl.when(c1 + 2 < n_i)
        def _():
            idx_src(c1 + 2, 1)

        return carry

    lax.fori_loop(0, (ECH2_T + ECH2_R + 1) // 2, body, 0)
    @pl.when(n_i == ECH2_T + 1)
    def _():
        wait_scatter(0)

    plsc.subcore_barrier()

    @pl.when(c == 0)
    def _():
        pltpu.sync_copy(acc_sh.at[pl.ds(base_r, RB)],
                        out_a.at[pl.ds(base_r, RB)])

    @pl.when(c == 1)
    def _():
        pltpu.sync_copy(acc_sh.at[pl.ds(base_r, RB)],
                        out_b.at[pl.ds(base_r, RB)])

    @pl.when((c == 0) & (s == NS - 1))
    def _():
        pltpu.sync_copy(acc_sh.at[pl.ds(N - 16, 16)],
                        out_a.at[pl.ds(N - 16, 16)])

    @pl.when((c == 1) & (s == NS - 1))
    def _():
        pltpu.sync_copy(acc_sh.at[pl.ds(N - 16, 16)],
                        out_b.at[pl.ds(N - 16, 16)])


CHP = 64
PCH = P // (NC * NS) // CHP
HALF = PCH // 2


@functools.partial(
    pl.kernel,
    out_type=(jax.ShapeDtypeStruct((P // 2,), jnp.float32),
              jax.ShapeDtypeStruct((P // 2, D), jnp.float32),
              jax.ShapeDtypeStruct((P // 2, D), jnp.float32)),
    mesh=_MESH,
    scratch_types=[
        pltpu.VMEM((2, CHP), jnp.int32),
        pltpu.VMEM((2, CHP), jnp.int32),
        pltpu.VMEM((2, CHP, D), jnp.float32),
        pltpu.VMEM((2, CHP, D), jnp.float32),
        pltpu.VMEM((2, CHP), jnp.float32),
        pltpu.SemaphoreType.DMA,
        pltpu.SemaphoreType.DMA,
        pltpu.SemaphoreType.DMA,
        pltpu.SemaphoreType.DMA,
        pltpu.SemaphoreType.DMA,
        pltpu.SemaphoreType.DMA,
    ],
)
def _sc_pairs(x2_hbm, pa_hbm, pb_hbm, out_hbm, drug_hbm, adr_hbm,
              ia_v, ib_v, ra_v, rb_v,
              sc_v, si0, si1, sa0, sa1, sb0, sb1):
    c = lax.axis_index("c")
    s = lax.axis_index("s")
    wid = c * NS + s
    lane = lax.iota(jnp.int32, L)
    sem_i = (si0, si1)
    sem_a = (sa0, sa1)
    sem_b = (sb0, sb1)

    def fetch_gather(ch, b):
        base = wid * (PCH * CHP) + ch * CHP
        pltpu.async_copy(pa_hbm.at[pl.ds(base, CHP)], ia_v.at[b], sem_i[b])
        pltpu.async_copy(pb_hbm.at[pl.ds(base, CHP)], ib_v.at[b], sem_i[b])
        pltpu.make_async_copy(pa_hbm.at[pl.ds(0, CHP)], ia_v.at[b],
                              sem_i[b]).wait()
        pltpu.make_async_copy(pa_hbm.at[pl.ds(0, CHP)], ib_v.at[b],
                              sem_i[b]).wait()
        pltpu.async_copy(x2_hbm.at[ia_v.at[b]], ra_v.at[b], sem_a[b])
        pltpu.async_copy(x2_hbm.at[ib_v.at[b]], rb_v.at[b], sem_b[b])

    perms = [jnp.bitwise_xor(lane, k) for k in (8, 4, 2, 1)]
    masks = [lane == pp for pp in range(L)]

    def compute(ch, b):
        pltpu.make_async_copy(x2_hbm.at[pl.ds(0, CHP)], ra_v.at[b],
                              sem_a[b]).wait()
        pltpu.make_async_copy(x2_hbm.at[pl.ds(0, CHP)], rb_v.at[b],
                              sem_b[b]).wait()

        @pl.when(ch >= HALF)
        def _():
            base_o = wid * (HALF * CHP) + (ch - HALF) * CHP
            pltpu.sync_copy(ra_v.at[b], drug_hbm.at[pl.ds(base_o, CHP)])
            pltpu.sync_copy(rb_v.at[b], adr_hbm.at[pl.ds(base_o, CHP)])

        @pl.when(ch < HALF)
        def _():
            dots(ch, b)

    def dots(ch, b):

        def butterfly(acc):
            for perm in perms:
                acc = acc + lax.gather(
                    acc, perm[:, None],
                    lax.GatherDimensionNumbers(
                        offset_dims=(), collapsed_slice_dims=(0,),
                        start_index_map=(0,)),
                    slice_sizes=(1,),
                    mode=lax.GatherScatterMode.PROMISE_IN_BOUNDS)
            return acc

        @plsc.parallel_loop(0, CHP // L, 1)
        def qbody(q):
            vec = jnp.zeros((L,), jnp.float32)
            for pp in range(L):
                p = q * L + pp
                z = jnp.zeros((L,), jnp.float32)
                partial = [z, z, z, z]
                for j in range(D // L):
                    partial[j % 4] = partial[j % 4] + (
                        ra_v[b, p, pl.ds(j * L, L)] *
                        rb_v[b, p, pl.ds(j * L, L)])
                acc = butterfly((partial[0] + partial[1]) +
                                (partial[2] + partial[3]))
                vec = jnp.where(masks[pp], acc, vec)
            sc_v[b, pl.ds(q * L, L)] = 1.0 / (1.0 + jnp.exp(-vec))

        base = wid * (HALF * CHP) + ch * CHP
        pltpu.sync_copy(sc_v.at[b], out_hbm.at[pl.ds(base, CHP)])

    fetch_gather(0, 0)
    fetch_gather(1, 1)

    def body(j, carry):
        compute(2 * j, 0)

        @pl.when(j < PCH // 2 - 1)
        def _():
            fetch_gather(2 * j + 2, 0)

        compute(2 * j + 1, 1)

        @pl.when(j < PCH // 2 - 1)
        def _():
            fetch_gather(2 * j + 3, 1)

        return carry

    lax.fori_loop(0, PCH // 2, body, 0)


BN = 2000
GRID = N // BN


def _dinv(degT_ref):
    d = degT_ref[:, 0:1] + degT_ref[:, 1:2] + 1.0
    return lax.rsqrt(d)


def _tc0_body(emb_ref, w1_ref, ha_ref, hb_ref):
    h = jnp.dot(emb_ref[...], w1_ref[...], preferred_element_type=jnp.float32)
    ha_ref[...] = h[:, :DH]
    hb_ref[...] = h[:, DH:]


def _tc1_body(degT_ref, ha_ref, hb_ref, xa_ref, xb_ref):
    dinv = _dinv(degT_ref)
    xa_ref[...] = ha_ref[...] * dinv
    xb_ref[...] = hb_ref[...] * dinv


def _tc2_body(degT_ref, sa_ref, sb_ref, xa_ref, xb_ref, b1_ref, w2_ref,
              ya_ref, yb_ref):
    dinv = _dinv(degT_ref)
    su = jnp.concatenate([sa_ref[...] + xa_ref[...],
                          sb_ref[...] + xb_ref[...]], axis=1)
    x1 = jnp.maximum(dinv * su + b1_ref[...], 0.0)
    h = jnp.dot(x1 * dinv, w2_ref[...], preferred_element_type=jnp.float32)
    ya_ref[...] = h[:, :DH]
    yb_ref[...] = h[:, DH:]


def _tc3_body(degT_ref, sa_ref, sb_ref, xa_ref, xb_ref, b2_ref, x2_ref):
    dinv = _dinv(degT_ref)
    su = jnp.concatenate([sa_ref[...] + xa_ref[...],
                          sb_ref[...] + xb_ref[...]], axis=1)
    x2_ref[...] = dinv * su + b2_ref[...]


def _tc_dot_body(a_ref, b_ref, o_ref):
    s = jnp.sum(a_ref[...] * b_ref[...], axis=2)
    o_ref[...] = 1.0 / (1.0 + jnp.exp(-s))


_tc_dot = pl.pallas_call(
    _tc_dot_body,
    grid=(8,),
    in_specs=[pl.BlockSpec((P // 2 // 128 // 8, 128, D), lambda i: (i, 0, 0)),
              pl.BlockSpec((P // 2 // 128 // 8, 128, D), lambda i: (i, 0, 0))],
    out_specs=pl.BlockSpec((P // 2 // 128 // 8, 128), lambda i: (i, 0)),
    out_shape=jax.ShapeDtypeStruct((P // 2 // 128, 128), jnp.float32),
)


def _rows(shape):
    return pl.BlockSpec(shape, lambda i: (i, 0))


def _full(shape):
    return pl.BlockSpec(shape, lambda i: (0, 0))


_tc0 = pl.pallas_call(
    _tc0_body,
    grid=(GRID,),
    in_specs=[_rows((BN, D)), _full((D, D))],
    out_specs=[_rows((BN, DH)), _rows((BN, DH))],
    out_shape=(jax.ShapeDtypeStruct((N, DH), jnp.float32),
               jax.ShapeDtypeStruct((N, DH), jnp.float32)),
)

_tc1 = pl.pallas_call(
    _tc1_body,
    grid=(GRID,),
    in_specs=[_rows((BN, 2)), _rows((BN, DH)), _rows((BN, DH))],
    out_specs=[_rows((BN, DH)), _rows((BN, DH))],
    out_shape=(jax.ShapeDtypeStruct((N, DH), jnp.float32),
               jax.ShapeDtypeStruct((N, DH), jnp.float32)),
)

_tc2 = pl.pallas_call(
    _tc2_body,
    grid=(GRID,),
    in_specs=[_rows((BN, 2)), _rows((BN, DH)), _rows((BN, DH)),
              _rows((BN, DH)), _rows((BN, DH)), _full((1, D)), _full((D, D))],
    out_specs=[_rows((BN, DH)), _rows((BN, DH))],
    out_shape=(jax.ShapeDtypeStruct((N, DH), jnp.float32),
               jax.ShapeDtypeStruct((N, DH), jnp.float32)),
)

_tc3 = pl.pallas_call(
    _tc3_body,
    grid=(GRID,),
    in_specs=[_rows((BN, 2)), _rows((BN, DH)), _rows((BN, DH)),
              _rows((BN, DH)), _rows((BN, DH)), _full((1, D))],
    out_specs=_rows((BN, D)),
    out_shape=jax.ShapeDtypeStruct((N, D), jnp.float32),
)


def kernel(edge_index, edge_pairs, emb, W1, b1, W2, b2):
    src = edge_index[0].astype(jnp.int32)
    dst = edge_index[1].astype(jnp.int32)
    pa = edge_pairs[:, 0].astype(jnp.int32)
    pb = edge_pairs[:, 1].astype(jnp.int32)

    zdeg = jnp.zeros((N,), jnp.float32)
    zrow = jnp.zeros((RB, DH), jnp.float32)

    src2 = src.reshape(E // 128, 128)
    dst2 = dst.reshape(E // 128, 128)

    deg2 = _sc_deg(dst, zdeg)
    degT = deg2.reshape(NC, N).T

    h0a, h0b = _tc0(emb, W1)
    xa0, xb0 = _tc1(degT, h0a, h0b)
    s1a, s1b = _sc_scatter(xa0, xb0, src2, dst2, zrow)
    ya, yb = _tc2(degT, s1a, s1b, xa0, xb0, b1.reshape(1, D), W2)
    s2a, s2b = _sc_scatter(ya, yb, src2, dst2, zrow)
    x2 = _tc3(degT, s2a, s2b, ya, yb, b2.reshape(1, D))

    out_sc, drug, adr = _sc_pairs(x2, pa, pb)
    out_tc = _tc_dot(drug.reshape(P // 2 // 128, 128, D),
                     adr.reshape(P // 2 // 128, 128, D))
    nw = NC * NS
    return jnp.concatenate(
        [out_sc.reshape(nw, HALF * CHP),
         out_tc.reshape(P // 2).reshape(nw, HALF * CHP)],
        axis=1).reshape(P)
